# baseline (speedup 1.0000x reference)
import functools

import jax
import jax.numpy as jnp
from jax import lax
from jax.experimental import pallas as pl
from jax.experimental.pallas import tpu as pltpu
from jax.experimental.pallas import tpu_sc as plsc

B = 8
N = 4096
L = 16
NC = 2
NS = 16
NW = NC * NS
CHUNKS = NW // B
CHUNK = N // CHUNKS
SBLK = 8
NT = N // L
NSV = CHUNK // L
W = 768
WB = W // L
GAMMA = 0.0045
INV2G = 1.0 / (2.0 * (1.0 - GAMMA))

_mesh = plsc.VectorSubcoreMesh(
    core_axis_name="c", subcore_axis_name="s", num_cores=NC, num_subcores=NS
)

_GATHER_DNUMS = lax.GatherDimensionNumbers(
    offset_dims=(), collapsed_slice_dims=(0,), start_index_map=(0,)
)


def _lane_bcast(v, k):
    idx = jnp.full((L, 1), k, jnp.int32)
    return lax.gather(
        v, idx, _GATHER_DNUMS, (1,),
        mode=lax.GatherScatterMode.PROMISE_IN_BOUNDS,
    )


def _bf16_rne(x):
    u = plsc.bitcast(x, jnp.uint32)
    r = u + jnp.uint32(0x7FFF) + ((u >> jnp.uint32(16)) & jnp.uint32(1))
    return plsc.bitcast(r & jnp.uint32(0xFFFF0000), jnp.float32)


def _sqrt(x):
    u = plsc.bitcast(x, jnp.uint32)
    y = plsc.bitcast((u >> jnp.uint32(1)) + jnp.uint32(0x1FBD1DF5), jnp.float32)
    for _ in range(3):
        y = 0.5 * (y + x / y)
    return y


@functools.partial(
    pl.kernel,
    mesh=_mesh,
    compiler_params=pltpu.CompilerParams(needs_layout_passes=False),
    out_type=jax.ShapeDtypeStruct((NW, 2, L), jnp.float32),
    scratch_types=[
        pltpu.VMEM((4, CHUNK), jnp.float32),
        pltpu.VMEM((4, N), jnp.float32),
        pltpu.VMEM((4, N), jnp.float32),
        pltpu.VMEM((2, L), jnp.float32),
    ],
)
def _sc_knn_loss(src_hbm, tgt_hbm, out_hbm, src_v, tgt_v, rw_v, out_v):
    wid = lax.axis_index("s") * NC + lax.axis_index("c")
    b = wid // CHUNKS
    cidx = wid % CHUNKS

    pltpu.sync_copy(src_hbm.at[wid], src_v)
    pltpu.sync_copy(tgt_hbm.at[b], tgt_v)

    inf_v = jnp.full((L,), jnp.inf, jnp.float32)
    zero_v = jnp.zeros((L,), jnp.float32)
    one_v = jnp.full((L,), 1.0, jnp.float32)
    iota_v = lax.iota(jnp.int32, L)
    big_i = jnp.full((L,), N, jnp.int32)
    zero_i = jnp.zeros((L,), jnp.int32)
    one_i = jnp.full((L,), 1, jnp.int32)
    two_i = jnp.full((L,), 2, jnp.int32)
    three_i = jnp.full((L,), 3, jnp.int32)

    def prep_body(j, _):
        o = j * L
        tx = tgt_v[0, pl.ds(o, L)]
        ty = tgt_v[1, pl.ds(o, L)]
        tz = tgt_v[2, pl.ds(o, L)]
        rw_v[0, pl.ds(o, L)] = 2.0 * _bf16_rne(tx)
        rw_v[1, pl.ds(o, L)] = 2.0 * _bf16_rne(ty)
        rw_v[2, pl.ds(o, L)] = 2.0 * _bf16_rne(tz)
        w = (tx * tx + ty * ty) + tz * tz
        valid = (tx != 0.0) | (ty != 0.0) | (tz != 0.0)
        rw_v[3, pl.ds(o, L)] = jnp.where(valid, w, inf_v)
        return 0

    lax.fori_loop(0, NT, prep_body, 0)

    def cnt_body(q, cta):
        o = q * L
        sx = src_v[0, pl.ds(o, L)]
        sy = src_v[1, pl.ds(o, L)]
        sz = src_v[2, pl.ds(o, L)]
        valid = (sx != 0.0) | (sy != 0.0) | (sz != 0.0)
        return cta + jnp.where(valid, one_v, zero_v)

    cta = lax.fori_loop(0, NSV, cnt_body, zero_v)

    def _searchpos(vbound, upper):
        base = zero_i
        for stride in (N // L, L, 1):
            idx = base + iota_v * jnp.int32(stride)
            vals = plsc.load_gather(tgt_v, [three_i, idx])
            m = (vals <= vbound) if upper else (vals < vbound)
            cnt = plsc.all_reduce_population_count(m)
            if stride == 1:
                base = base + cnt
            else:
                base = base + jnp.maximum(cnt - 1, 0) * jnp.int32(stride)
        return base

    def grp_body(q, dacc):
        o = q * L
        sxv = src_v[0, pl.ds(o, L)]
        syv = src_v[1, pl.ds(o, L)]
        szv = src_v[2, pl.ds(o, L)]
        rsx = _bf16_rne(sxv)
        rsy = _bf16_rne(syv)
        rsz = _bf16_rne(szv)
        for half in range(L // SBLK):
            hb = half * SBLK
            bx = [_lane_bcast(rsx, hb + k) for k in range(SBLK)]
            by = [_lane_bcast(rsy, hb + k) for k in range(SBLK)]
            bz = [_lane_bcast(rsz, hb + k) for k in range(SBLK)]

            def tgt_body(jb, carry):
                mv = carry[:SBLK]
                mi = carry[SBLK:]
                to = jb * L
                rtx = rw_v[0, pl.ds(to, L)]
                rty = rw_v[1, pl.ds(to, L)]
                rtz = rw_v[2, pl.ds(to, L)]
                w = rw_v[3, pl.ds(to, L)]
                jvec = iota_v + to
                nmv = []
                nmi = []
                for k in range(SBLK):
                    v = w - ((bx[k] * rtx + by[k] * rty) + bz[k] * rtz)
                    cmp = v < mv[k]
                    nmv.append(jnp.where(cmp, v, mv[k]))
                    nmi.append(jnp.where(cmp, jvec, mi[k]))
                return (*nmv, *nmi)

            center = CHUNKS * (o + hb + SBLK // 2) + cidx
            lo = jnp.clip(center - W // 2, 0, N - W)
            lo_b = lo // L
            init = (inf_v,) * SBLK + (zero_i,) * SBLK
            res = lax.fori_loop(lo_b, lo_b + WB, tgt_body, init)

            gmv = inf_v
            for k in range(SBLK):
                gm = -_lane_bcast(plsc.cummax(-res[k]), L - 1)
                gmv = jnp.where(iota_v == (hb + k), gm, gmv)
            nrs_vec = src_v[3, pl.ds(o, L)]
            c = 2.0 * nrs_vec * 1.00001
            disc = c * c + (4.0 * (1.0 - GAMMA)) * gmv
            disc = jnp.minimum(jnp.maximum(disc, 1e-12), 1e30)
            sq = _sqrt(disc)
            xlo = (c - sq) * INV2G - 1e-4
            xhi = (c + sq) * INV2G + 1e-4
            inhalf = (iota_v >= hb) & (iota_v < hb + SBLK)
            xlo = jnp.where(inhalf, xlo, jnp.full((L,), 1e30, jnp.float32))
            xhi = jnp.where(inhalf, xhi, jnp.full((L,), -1e30, jnp.float32))
            vlo = -_lane_bcast(plsc.cummax(-xlo), L - 1)
            vhi = _lane_bcast(plsc.cummax(xhi), L - 1)

            plo_v = _searchpos(vlo, upper=False)
            phi_v = _searchpos(vhi, upper=True)
            plo_b = (jnp.maximum(plo_v - L, 0) // jnp.int32(L))[0]
            phi_pad = jnp.minimum(phi_v + L, jnp.int32(N))
            phi_b = ((phi_pad + jnp.int32(L - 1)) // jnp.int32(L))[0]

            res = lax.fori_loop(plo_b, jnp.minimum(lo_b, phi_b), tgt_body, res)
            res = lax.fori_loop(jnp.maximum(lo_b + WB, plo_b), phi_b, tgt_body, res)
            mv = res[:SBLK]
            mi = res[SBLK:]

            osx = src_v[0, pl.ds(o, L)]
            osy = src_v[1, pl.ds(o, L)]
            osz = src_v[2, pl.ds(o, L)]
            for k in range(SBLK):
                gm = -_lane_bcast(plsc.cummax(-mv[k]), L - 1)
                cand = jnp.where(mv[k] == gm, mi[k], big_i)
                idx = -_lane_bcast(plsc.cummax(-cand), L - 1)
                gx = plsc.load_gather(tgt_v, [zero_i, idx])
                gy = plsc.load_gather(tgt_v, [one_i, idx])
                gz = plsc.load_gather(tgt_v, [two_i, idx])
                ox = _lane_bcast(osx, hb + k)
                oy = _lane_bcast(osy, hb + k)
                oz = _lane_bcast(osz, hb + k)
                dx = ox - gx
                dy = oy - gy
                dz = oz - gz
                dd = (dx * dx + dy * dy) + dz * dz
                svalid = (ox != 0.0) | (oy != 0.0) | (oz != 0.0)
                dacc = dacc + jnp.where(svalid, dd, zero_v)
        return dacc

    dacc = lax.fori_loop(0, NSV, grp_body, zero_v)

    out_v[0, :] = dacc
    out_v[1, :] = cta
    pltpu.sync_copy(out_v, out_hbm.at[wid])


def kernel(source_point_cloud, target_point_cloud):
    srcT = source_point_cloud.astype(jnp.float32)
    tgtT = target_point_cloud.astype(jnp.float32)

    def norm_sort(p):
        r = p.astype(jnp.bfloat16).astype(jnp.float32)
        n = jnp.sqrt(jnp.sum(r * r, axis=-1))
        order = jnp.argsort(n, axis=1)
        ps = jnp.take_along_axis(p, order[..., None], axis=1)
        ns = jnp.take_along_axis(n, order, axis=1)
        return jnp.concatenate([ps.transpose(0, 2, 1), ns[:, None, :]], axis=1)

    src4 = norm_sort(srcT)
    tgt4 = norm_sort(tgtT)
    src_chunks = (
        src4.reshape(B, 4, CHUNK, CHUNKS).transpose(0, 3, 1, 2).reshape(NW, 4, CHUNK)
    )
    part = _sc_knn_loss(src_chunks, tgt4)

    d2sum = part[:, 0, 0]
    cnt = part[:, 1, :].sum(-1)
    loss_b = d2sum.reshape(B, CHUNKS).sum(1) / (3.0 * cnt.reshape(B, CHUNKS).sum(1))
    return jnp.mean(loss_b)


# speedup vs baseline: 1.0169x; 1.0169x over previous
import functools

import jax
import jax.numpy as jnp
from jax import lax
from jax.experimental import pallas as pl
from jax.experimental.pallas import tpu as pltpu
from jax.experimental.pallas import tpu_sc as plsc

B = 8
N = 4096
L = 16
NC = 2
NS = 16
NW = NC * NS
CHUNKS = NW // B
CHUNK = N // CHUNKS
SBLK = 8
NT = N // L
NSV = CHUNK // L
W = 640
WB = W // L
GAMMA = 0.0045
INV2G = 1.0 / (2.0 * (1.0 - GAMMA))

_mesh = plsc.VectorSubcoreMesh(
    core_axis_name="c", subcore_axis_name="s", num_cores=NC, num_subcores=NS
)

_GATHER_DNUMS = lax.GatherDimensionNumbers(
    offset_dims=(), collapsed_slice_dims=(0,), start_index_map=(0,)
)


def _lane_bcast(v, k):
    idx = jnp.full((L, 1), k, jnp.int32)
    return lax.gather(
        v, idx, _GATHER_DNUMS, (1,),
        mode=lax.GatherScatterMode.PROMISE_IN_BOUNDS,
    )


def _bf16_rne(x):
    u = plsc.bitcast(x, jnp.uint32)
    r = u + jnp.uint32(0x7FFF) + ((u >> jnp.uint32(16)) & jnp.uint32(1))
    return plsc.bitcast(r & jnp.uint32(0xFFFF0000), jnp.float32)


def _sqrt(x):
    u = plsc.bitcast(x, jnp.uint32)
    y = plsc.bitcast((u >> jnp.uint32(1)) + jnp.uint32(0x1FBD1DF5), jnp.float32)
    for _ in range(3):
        y = 0.5 * (y + x / y)
    return y


@functools.partial(
    pl.kernel,
    mesh=_mesh,
    compiler_params=pltpu.CompilerParams(needs_layout_passes=False),
    out_type=jax.ShapeDtypeStruct((NW, 2, L), jnp.float32),
    scratch_types=[
        pltpu.VMEM((4, CHUNK), jnp.float32),
        pltpu.VMEM((4, N), jnp.float32),
        pltpu.VMEM((4, N), jnp.float32),
        pltpu.VMEM((2, L), jnp.float32),
    ],
)
def _sc_knn_loss(src_hbm, tgt_hbm, out_hbm, src_v, tgt_v, rw_v, out_v):
    wid = lax.axis_index("s") * NC + lax.axis_index("c")
    b = wid // CHUNKS
    cidx = wid % CHUNKS

    pltpu.sync_copy(src_hbm.at[wid], src_v)
    pltpu.sync_copy(tgt_hbm.at[b], tgt_v)

    inf_v = jnp.full((L,), jnp.inf, jnp.float32)
    zero_v = jnp.zeros((L,), jnp.float32)
    one_v = jnp.full((L,), 1.0, jnp.float32)
    iota_v = lax.iota(jnp.int32, L)
    big_i = jnp.full((L,), N, jnp.int32)
    zero_i = jnp.zeros((L,), jnp.int32)
    one_i = jnp.full((L,), 1, jnp.int32)
    two_i = jnp.full((L,), 2, jnp.int32)
    three_i = jnp.full((L,), 3, jnp.int32)

    def prep_body(j, _):
        o = j * L
        tx = tgt_v[0, pl.ds(o, L)]
        ty = tgt_v[1, pl.ds(o, L)]
        tz = tgt_v[2, pl.ds(o, L)]
        rw_v[0, pl.ds(o, L)] = 2.0 * _bf16_rne(tx)
        rw_v[1, pl.ds(o, L)] = 2.0 * _bf16_rne(ty)
        rw_v[2, pl.ds(o, L)] = 2.0 * _bf16_rne(tz)
        w = (tx * tx + ty * ty) + tz * tz
        valid = (tx != 0.0) | (ty != 0.0) | (tz != 0.0)
        rw_v[3, pl.ds(o, L)] = jnp.where(valid, w, inf_v)
        return 0

    lax.fori_loop(0, NT, prep_body, 0)

    def cnt_body(q, cta):
        o = q * L
        sx = src_v[0, pl.ds(o, L)]
        sy = src_v[1, pl.ds(o, L)]
        sz = src_v[2, pl.ds(o, L)]
        valid = (sx != 0.0) | (sy != 0.0) | (sz != 0.0)
        return cta + jnp.where(valid, one_v, zero_v)

    cta = lax.fori_loop(0, NSV, cnt_body, zero_v)

    def _searchpos(vbound, upper):
        base = zero_i
        for stride in (N // L, L, 1):
            idx = base + iota_v * jnp.int32(stride)
            vals = plsc.load_gather(tgt_v, [three_i, idx])
            m = (vals <= vbound) if upper else (vals < vbound)
            cnt = plsc.all_reduce_population_count(m)
            if stride == 1:
                base = base + cnt
            else:
                base = base + jnp.maximum(cnt - 1, 0) * jnp.int32(stride)
        return base

    def grp_body(q, dacc):
        o = q * L
        sxv = src_v[0, pl.ds(o, L)]
        syv = src_v[1, pl.ds(o, L)]
        szv = src_v[2, pl.ds(o, L)]
        rsx = _bf16_rne(sxv)
        rsy = _bf16_rne(syv)
        rsz = _bf16_rne(szv)
        for half in range(L // SBLK):
            hb = half * SBLK
            bx = [_lane_bcast(rsx, hb + k) for k in range(SBLK)]
            by = [_lane_bcast(rsy, hb + k) for k in range(SBLK)]
            bz = [_lane_bcast(rsz, hb + k) for k in range(SBLK)]

            def tgt_body(jb, carry):
                mv = carry[:SBLK]
                mi = carry[SBLK:]
                to = jb * L
                rtx = rw_v[0, pl.ds(to, L)]
                rty = rw_v[1, pl.ds(to, L)]
                rtz = rw_v[2, pl.ds(to, L)]
                w = rw_v[3, pl.ds(to, L)]
                jvec = iota_v + to
                nmv = []
                nmi = []
                for k in range(SBLK):
                    v = w - ((bx[k] * rtx + by[k] * rty) + bz[k] * rtz)
                    cmp = v < mv[k]
                    nmv.append(jnp.where(cmp, v, mv[k]))
                    nmi.append(jnp.where(cmp, jvec, mi[k]))
                return (*nmv, *nmi)

            center = cidx * CHUNK + o + hb + SBLK // 2
            lo = jnp.clip(center - W // 2, 0, N - W)
            lo_b = lo // L
            init = (inf_v,) * SBLK + (zero_i,) * SBLK
            res = lax.fori_loop(lo_b, lo_b + WB, tgt_body, init)

            gmv = inf_v
            for k in range(SBLK):
                gm = -_lane_bcast(plsc.cummax(-res[k]), L - 1)
                gmv = jnp.where(iota_v == (hb + k), gm, gmv)
            nrs_vec = src_v[3, pl.ds(o, L)]
            c = 2.0 * nrs_vec * 1.00001
            disc = c * c + (4.0 * (1.0 - GAMMA)) * gmv
            disc = jnp.minimum(jnp.maximum(disc, 1e-12), 1e30)
            sq = _sqrt(disc)
            xlo = (c - sq) * INV2G - 1e-4
            xhi = (c + sq) * INV2G + 1e-4
            inhalf = (iota_v >= hb) & (iota_v < hb + SBLK)
            xlo = jnp.where(inhalf, xlo, jnp.full((L,), 1e30, jnp.float32))
            xhi = jnp.where(inhalf, xhi, jnp.full((L,), -1e30, jnp.float32))
            vlo = -_lane_bcast(plsc.cummax(-xlo), L - 1)
            vhi = _lane_bcast(plsc.cummax(xhi), L - 1)

            plo_v = _searchpos(vlo, upper=False)
            phi_v = _searchpos(vhi, upper=True)
            plo_b = (jnp.maximum(plo_v - L, 0) // jnp.int32(L))[0]
            phi_pad = jnp.minimum(phi_v + L, jnp.int32(N))
            phi_b = ((phi_pad + jnp.int32(L - 1)) // jnp.int32(L))[0]

            res = lax.fori_loop(plo_b, jnp.minimum(lo_b, phi_b), tgt_body, res)
            res = lax.fori_loop(jnp.maximum(lo_b + WB, plo_b), phi_b, tgt_body, res)
            mv = res[:SBLK]
            mi = res[SBLK:]

            osx = src_v[0, pl.ds(o, L)]
            osy = src_v[1, pl.ds(o, L)]
            osz = src_v[2, pl.ds(o, L)]
            for k in range(SBLK):
                gm = -_lane_bcast(plsc.cummax(-mv[k]), L - 1)
                cand = jnp.where(mv[k] == gm, mi[k], big_i)
                idx = -_lane_bcast(plsc.cummax(-cand), L - 1)
                gx = plsc.load_gather(tgt_v, [zero_i, idx])
                gy = plsc.load_gather(tgt_v, [one_i, idx])
                gz = plsc.load_gather(tgt_v, [two_i, idx])
                ox = _lane_bcast(osx, hb + k)
                oy = _lane_bcast(osy, hb + k)
                oz = _lane_bcast(osz, hb + k)
                dx = ox - gx
                dy = oy - gy
                dz = oz - gz
                dd = (dx * dx + dy * dy) + dz * dz
                svalid = (ox != 0.0) | (oy != 0.0) | (oz != 0.0)
                dacc = dacc + jnp.where(svalid, dd, zero_v)
        return dacc

    dacc = lax.fori_loop(0, NSV, grp_body, zero_v)

    out_v[0, :] = dacc
    out_v[1, :] = cta
    pltpu.sync_copy(out_v, out_hbm.at[wid])


def kernel(source_point_cloud, target_point_cloud):
    srcT = source_point_cloud.astype(jnp.float32)
    tgtT = target_point_cloud.astype(jnp.float32)

    def norm_sort(p):
        r = p.astype(jnp.bfloat16).astype(jnp.float32)
        n = jnp.sqrt(jnp.sum(r * r, axis=-1))
        order = jnp.argsort(n, axis=1)
        ps = jnp.take_along_axis(p, order[..., None], axis=1)
        ns = jnp.take_along_axis(n, order, axis=1)
        return jnp.concatenate([ps.transpose(0, 2, 1), ns[:, None, :]], axis=1)

    src4 = norm_sort(srcT)
    tgt4 = norm_sort(tgtT)
    src_chunks = (
        src4.reshape(B, 4, CHUNKS, CHUNK).transpose(0, 2, 1, 3).reshape(NW, 4, CHUNK)
    )
    part = _sc_knn_loss(src_chunks, tgt4)

    d2sum = part[:, 0, 0]
    cnt = part[:, 1, :].sum(-1)
    loss_b = d2sum.reshape(B, CHUNKS).sum(1) / (3.0 * cnt.reshape(B, CHUNKS).sum(1))
    return jnp.mean(loss_b)


# speedup vs baseline: 1.0463x; 1.0289x over previous
import functools

import jax
import jax.numpy as jnp
from jax import lax
from jax.experimental import pallas as pl
from jax.experimental.pallas import tpu as pltpu
from jax.experimental.pallas import tpu_sc as plsc

B = 8
N = 4096
L = 16
NC = 2
NS = 16
NW = NC * NS
CHUNKS = NW // B
CHUNK = N // CHUNKS
SBLK = 8
NT = N // L
NSV = CHUNK // L
W = 896
WB = W // L
GAMMA = 0.0045
INV2G = 1.0 / (2.0 * (1.0 - GAMMA))

_mesh = plsc.VectorSubcoreMesh(
    core_axis_name="c", subcore_axis_name="s", num_cores=NC, num_subcores=NS
)

_GATHER_DNUMS = lax.GatherDimensionNumbers(
    offset_dims=(), collapsed_slice_dims=(0,), start_index_map=(0,)
)


def _lane_bcast(v, k):
    idx = jnp.full((L, 1), k, jnp.int32)
    return lax.gather(
        v, idx, _GATHER_DNUMS, (1,),
        mode=lax.GatherScatterMode.PROMISE_IN_BOUNDS,
    )


def _bf16_rne(x):
    u = plsc.bitcast(x, jnp.uint32)
    r = u + jnp.uint32(0x7FFF) + ((u >> jnp.uint32(16)) & jnp.uint32(1))
    return plsc.bitcast(r & jnp.uint32(0xFFFF0000), jnp.float32)


def _sqrt(x):
    u = plsc.bitcast(x, jnp.uint32)
    y = plsc.bitcast((u >> jnp.uint32(1)) + jnp.uint32(0x1FBD1DF5), jnp.float32)
    for _ in range(3):
        y = 0.5 * (y + x / y)
    return y


@functools.partial(
    pl.kernel,
    mesh=_mesh,
    compiler_params=pltpu.CompilerParams(needs_layout_passes=False),
    out_type=jax.ShapeDtypeStruct((NW, 2, L), jnp.float32),
    scratch_types=[
        pltpu.VMEM((4, CHUNK), jnp.float32),
        pltpu.VMEM((4, N), jnp.float32),
        pltpu.VMEM((4, N), jnp.float32),
        pltpu.VMEM((2, L), jnp.float32),
    ],
)
def _sc_knn_loss(src_hbm, tgt_hbm, out_hbm, src_v, tgt_v, rw_v, out_v):
    wid = lax.axis_index("s") * NC + lax.axis_index("c")
    b = wid // CHUNKS
    cidx = wid % CHUNKS

    pltpu.sync_copy(src_hbm.at[wid], src_v)
    pltpu.sync_copy(tgt_hbm.at[b], tgt_v)

    inf_v = jnp.full((L,), jnp.inf, jnp.float32)
    zero_v = jnp.zeros((L,), jnp.float32)
    one_v = jnp.full((L,), 1.0, jnp.float32)
    iota_v = lax.iota(jnp.int32, L)
    big_i = jnp.full((L,), N, jnp.int32)
    zero_i = jnp.zeros((L,), jnp.int32)
    one_i = jnp.full((L,), 1, jnp.int32)
    two_i = jnp.full((L,), 2, jnp.int32)
    three_i = jnp.full((L,), 3, jnp.int32)

    def prep_body(j, _):
        o = j * L
        tx = tgt_v[0, pl.ds(o, L)]
        ty = tgt_v[1, pl.ds(o, L)]
        tz = tgt_v[2, pl.ds(o, L)]
        rw_v[0, pl.ds(o, L)] = 2.0 * _bf16_rne(tx)
        rw_v[1, pl.ds(o, L)] = 2.0 * _bf16_rne(ty)
        rw_v[2, pl.ds(o, L)] = 2.0 * _bf16_rne(tz)
        w = (tx * tx + ty * ty) + tz * tz
        valid = (tx != 0.0) | (ty != 0.0) | (tz != 0.0)
        rw_v[3, pl.ds(o, L)] = jnp.where(valid, w, inf_v)
        return 0

    lax.fori_loop(0, NT, prep_body, 0)

    def cnt_body(q, cta):
        o = q * L
        sx = src_v[0, pl.ds(o, L)]
        sy = src_v[1, pl.ds(o, L)]
        sz = src_v[2, pl.ds(o, L)]
        valid = (sx != 0.0) | (sy != 0.0) | (sz != 0.0)
        return cta + jnp.where(valid, one_v, zero_v)

    cta = lax.fori_loop(0, NSV, cnt_body, zero_v)

    def _searchpos(vbound, upper):
        base = zero_i
        for stride in (N // L, L, 1):
            idx = base + iota_v * jnp.int32(stride)
            vals = plsc.load_gather(tgt_v, [three_i, idx])
            m = (vals <= vbound) if upper else (vals < vbound)
            cnt = plsc.all_reduce_population_count(m)
            if stride == 1:
                base = base + cnt
            else:
                base = base + jnp.maximum(cnt - 1, 0) * jnp.int32(stride)
        return base

    def grp_body(q, dacc):
        o = q * L
        sxv = src_v[0, pl.ds(o, L)]
        syv = src_v[1, pl.ds(o, L)]
        szv = src_v[2, pl.ds(o, L)]
        rsx = _bf16_rne(sxv)
        rsy = _bf16_rne(syv)
        rsz = _bf16_rne(szv)
        for half in range(L // SBLK):
            hb = half * SBLK
            bx = [_lane_bcast(rsx, hb + k) for k in range(SBLK)]
            by = [_lane_bcast(rsy, hb + k) for k in range(SBLK)]
            bz = [_lane_bcast(rsz, hb + k) for k in range(SBLK)]

            def tgt_body(jb, carry):
                mv = carry[:SBLK]
                mi = carry[SBLK:]
                to = jb * L
                rtx = rw_v[0, pl.ds(to, L)]
                rty = rw_v[1, pl.ds(to, L)]
                rtz = rw_v[2, pl.ds(to, L)]
                w = rw_v[3, pl.ds(to, L)]
                jvec = iota_v + to
                nmv = []
                nmi = []
                for k in range(SBLK):
                    v = w - ((bx[k] * rtx + by[k] * rty) + bz[k] * rtz)
                    cmp = v < mv[k]
                    nmv.append(jnp.where(cmp, v, mv[k]))
                    nmi.append(jnp.where(cmp, jvec, mi[k]))
                return (*nmv, *nmi)

            center = cidx * CHUNK + o + hb + SBLK // 2
            lo = jnp.clip(center - W // 2, 0, N - W)
            lo_b = lo // L
            init = (inf_v,) * SBLK + (zero_i,) * SBLK
            res = lax.fori_loop(lo_b, lo_b + WB, tgt_body, init)

            gmv = inf_v
            for k in range(SBLK):
                gm = -_lane_bcast(plsc.cummax(-res[k]), L - 1)
                gmv = jnp.where(iota_v == (hb + k), gm, gmv)
            nrs_vec = src_v[3, pl.ds(o, L)]
            c = 2.0 * nrs_vec * 1.00001
            disc = c * c + (4.0 * (1.0 - GAMMA)) * gmv
            disc = jnp.minimum(jnp.maximum(disc, 1e-12), 1e30)
            sq = _sqrt(disc)
            xlo = (c - sq) * INV2G - 1e-4
            xhi = (c + sq) * INV2G + 1e-4
            inhalf = (iota_v >= hb) & (iota_v < hb + SBLK)
            xlo = jnp.where(inhalf, xlo, jnp.full((L,), 1e30, jnp.float32))
            xhi = jnp.where(inhalf, xhi, jnp.full((L,), -1e30, jnp.float32))
            vlo = -_lane_bcast(plsc.cummax(-xlo), L - 1)
            vhi = _lane_bcast(plsc.cummax(xhi), L - 1)

            plo_v = _searchpos(vlo, upper=False)
            phi_v = _searchpos(vhi, upper=True)
            plo_b = (jnp.maximum(plo_v - L, 0) // jnp.int32(L))[0]
            phi_pad = jnp.minimum(phi_v + L, jnp.int32(N))
            phi_b = ((phi_pad + jnp.int32(L - 1)) // jnp.int32(L))[0]

            res = lax.fori_loop(plo_b, jnp.minimum(lo_b, phi_b), tgt_body, res)
            res = lax.fori_loop(jnp.maximum(lo_b + WB, plo_b), phi_b, tgt_body, res)
            mv = res[:SBLK]
            mi = res[SBLK:]

            osx = src_v[0, pl.ds(o, L)]
            osy = src_v[1, pl.ds(o, L)]
            osz = src_v[2, pl.ds(o, L)]
            for k in range(SBLK):
                gm = -_lane_bcast(plsc.cummax(-mv[k]), L - 1)
                cand = jnp.where(mv[k] == gm, mi[k], big_i)
                idx = -_lane_bcast(plsc.cummax(-cand), L - 1)
                gx = plsc.load_gather(tgt_v, [zero_i, idx])
                gy = plsc.load_gather(tgt_v, [one_i, idx])
                gz = plsc.load_gather(tgt_v, [two_i, idx])
                ox = _lane_bcast(osx, hb + k)
                oy = _lane_bcast(osy, hb + k)
                oz = _lane_bcast(osz, hb + k)
                dx = ox - gx
                dy = oy - gy
                dz = oz - gz
                dd = (dx * dx + dy * dy) + dz * dz
                svalid = (ox != 0.0) | (oy != 0.0) | (oz != 0.0)
                dacc = dacc + jnp.where(svalid, dd, zero_v)
        return dacc

    dacc = lax.fori_loop(0, NSV, grp_body, zero_v)

    out_v[0, :] = dacc
    out_v[1, :] = cta
    pltpu.sync_copy(out_v, out_hbm.at[wid])


def kernel(source_point_cloud, target_point_cloud):
    srcT = source_point_cloud.astype(jnp.float32)
    tgtT = target_point_cloud.astype(jnp.float32)

    def norm_sort(p):
        r = p.astype(jnp.bfloat16).astype(jnp.float32)
        n = jnp.sqrt(jnp.sum(r * r, axis=-1))
        order = jnp.argsort(n, axis=1)
        ps = jnp.take_along_axis(p, order[..., None], axis=1)
        ns = jnp.take_along_axis(n, order, axis=1)
        return jnp.concatenate([ps.transpose(0, 2, 1), ns[:, None, :]], axis=1)

    src4 = norm_sort(srcT)
    tgt4 = norm_sort(tgtT)
    src_chunks = (
        src4.reshape(B, 4, CHUNKS, CHUNK).transpose(0, 2, 1, 3).reshape(NW, 4, CHUNK)
    )
    part = _sc_knn_loss(src_chunks, tgt4)

    d2sum = part[:, 0, 0]
    cnt = part[:, 1, :].sum(-1)
    loss_b = d2sum.reshape(B, CHUNKS).sum(1) / (3.0 * cnt.reshape(B, CHUNKS).sum(1))
    return jnp.mean(loss_b)


# speedup vs baseline: 1.0474x; 1.0010x over previous
import functools

import jax
import jax.numpy as jnp
from jax import lax
from jax.experimental import pallas as pl
from jax.experimental.pallas import tpu as pltpu
from jax.experimental.pallas import tpu_sc as plsc

B = 8
N = 4096
L = 16
NC = 2
NS = 16
NW = NC * NS
CHUNKS = NW // B
CHUNK = N // CHUNKS
SBLK = 8
NT = N // L
NSV = CHUNK // L
W = 1024
WB = W // L
GAMMA = 0.0045
INV2G = 1.0 / (2.0 * (1.0 - GAMMA))

_mesh = plsc.VectorSubcoreMesh(
    core_axis_name="c", subcore_axis_name="s", num_cores=NC, num_subcores=NS
)

_GATHER_DNUMS = lax.GatherDimensionNumbers(
    offset_dims=(), collapsed_slice_dims=(0,), start_index_map=(0,)
)


def _lane_bcast(v, k):
    idx = jnp.full((L, 1), k, jnp.int32)
    return lax.gather(
        v, idx, _GATHER_DNUMS, (1,),
        mode=lax.GatherScatterMode.PROMISE_IN_BOUNDS,
    )


def _bf16_rne(x):
    u = plsc.bitcast(x, jnp.uint32)
    r = u + jnp.uint32(0x7FFF) + ((u >> jnp.uint32(16)) & jnp.uint32(1))
    return plsc.bitcast(r & jnp.uint32(0xFFFF0000), jnp.float32)


def _sqrt(x):
    u = plsc.bitcast(x, jnp.uint32)
    y = plsc.bitcast((u >> jnp.uint32(1)) + jnp.uint32(0x1FBD1DF5), jnp.float32)
    for _ in range(3):
        y = 0.5 * (y + x / y)
    return y


@functools.partial(
    pl.kernel,
    mesh=_mesh,
    compiler_params=pltpu.CompilerParams(needs_layout_passes=False),
    out_type=jax.ShapeDtypeStruct((NW, 2, L), jnp.float32),
    scratch_types=[
        pltpu.VMEM((4, CHUNK), jnp.float32),
        pltpu.VMEM((4, N), jnp.float32),
        pltpu.VMEM((4, N), jnp.float32),
        pltpu.VMEM((2, L), jnp.float32),
    ],
)
def _sc_knn_loss(src_hbm, tgt_hbm, out_hbm, src_v, tgt_v, rw_v, out_v):
    wid = lax.axis_index("s") * NC + lax.axis_index("c")
    b = wid // CHUNKS
    cidx = wid % CHUNKS

    pltpu.sync_copy(src_hbm.at[wid], src_v)
    pltpu.sync_copy(tgt_hbm.at[b], tgt_v)

    inf_v = jnp.full((L,), jnp.inf, jnp.float32)
    zero_v = jnp.zeros((L,), jnp.float32)
    one_v = jnp.full((L,), 1.0, jnp.float32)
    iota_v = lax.iota(jnp.int32, L)
    big_i = jnp.full((L,), N, jnp.int32)
    zero_i = jnp.zeros((L,), jnp.int32)
    one_i = jnp.full((L,), 1, jnp.int32)
    two_i = jnp.full((L,), 2, jnp.int32)
    three_i = jnp.full((L,), 3, jnp.int32)

    def prep_body(j, _):
        o = j * L
        tx = tgt_v[0, pl.ds(o, L)]
        ty = tgt_v[1, pl.ds(o, L)]
        tz = tgt_v[2, pl.ds(o, L)]
        rw_v[0, pl.ds(o, L)] = 2.0 * _bf16_rne(tx)
        rw_v[1, pl.ds(o, L)] = 2.0 * _bf16_rne(ty)
        rw_v[2, pl.ds(o, L)] = 2.0 * _bf16_rne(tz)
        w = (tx * tx + ty * ty) + tz * tz
        valid = (tx != 0.0) | (ty != 0.0) | (tz != 0.0)
        rw_v[3, pl.ds(o, L)] = jnp.where(valid, w, inf_v)
        return 0

    lax.fori_loop(0, NT, prep_body, 0)

    def cnt_body(q, cta):
        o = q * L
        sx = src_v[0, pl.ds(o, L)]
        sy = src_v[1, pl.ds(o, L)]
        sz = src_v[2, pl.ds(o, L)]
        valid = (sx != 0.0) | (sy != 0.0) | (sz != 0.0)
        return cta + jnp.where(valid, one_v, zero_v)

    cta = lax.fori_loop(0, NSV, cnt_body, zero_v)

    def _searchpos(vbound, upper):
        base = zero_i
        for stride in (N // L, L, 1):
            idx = base + iota_v * jnp.int32(stride)
            vals = plsc.load_gather(tgt_v, [three_i, idx])
            m = (vals <= vbound) if upper else (vals < vbound)
            cnt = plsc.all_reduce_population_count(m)
            if stride == 1:
                base = base + cnt
            else:
                base = base + jnp.maximum(cnt - 1, 0) * jnp.int32(stride)
        return base

    def grp_body(q, dacc):
        o = q * L
        sxv = src_v[0, pl.ds(o, L)]
        syv = src_v[1, pl.ds(o, L)]
        szv = src_v[2, pl.ds(o, L)]
        rsx = _bf16_rne(sxv)
        rsy = _bf16_rne(syv)
        rsz = _bf16_rne(szv)
        for half in range(L // SBLK):
            hb = half * SBLK
            bx = [_lane_bcast(rsx, hb + k) for k in range(SBLK)]
            by = [_lane_bcast(rsy, hb + k) for k in range(SBLK)]
            bz = [_lane_bcast(rsz, hb + k) for k in range(SBLK)]

            def tgt_body(jb, carry):
                mv = carry[:SBLK]
                mi = carry[SBLK:]
                to = jb * L
                rtx = rw_v[0, pl.ds(to, L)]
                rty = rw_v[1, pl.ds(to, L)]
                rtz = rw_v[2, pl.ds(to, L)]
                w = rw_v[3, pl.ds(to, L)]
                jvec = iota_v + to
                nmv = []
                nmi = []
                for k in range(SBLK):
                    v = w - ((bx[k] * rtx + by[k] * rty) + bz[k] * rtz)
                    cmp = v < mv[k]
                    nmv.append(jnp.where(cmp, v, mv[k]))
                    nmi.append(jnp.where(cmp, jvec, mi[k]))
                return (*nmv, *nmi)

            center = cidx * CHUNK + o + hb + SBLK // 2
            lo = jnp.clip(center - W // 2, 0, N - W)
            lo_b = lo // L
            init = (inf_v,) * SBLK + (zero_i,) * SBLK
            res = lax.fori_loop(lo_b, lo_b + WB, tgt_body, init)

            gmv = inf_v
            for k in range(SBLK):
                gm = -_lane_bcast(plsc.cummax(-res[k]), L - 1)
                gmv = jnp.where(iota_v == (hb + k), gm, gmv)
            nrs_vec = src_v[3, pl.ds(o, L)]
            c = 2.0 * nrs_vec * 1.00001
            disc = c * c + (4.0 * (1.0 - GAMMA)) * gmv
            disc = jnp.minimum(jnp.maximum(disc, 1e-12), 1e30)
            sq = _sqrt(disc)
            xlo = (c - sq) * INV2G - 1e-4
            xhi = (c + sq) * INV2G + 1e-4
            inhalf = (iota_v >= hb) & (iota_v < hb + SBLK)
            xlo = jnp.where(inhalf, xlo, jnp.full((L,), 1e30, jnp.float32))
            xhi = jnp.where(inhalf, xhi, jnp.full((L,), -1e30, jnp.float32))
            vlo = -_lane_bcast(plsc.cummax(-xlo), L - 1)
            vhi = _lane_bcast(plsc.cummax(xhi), L - 1)

            plo_v = _searchpos(vlo, upper=False)
            phi_v = _searchpos(vhi, upper=True)
            plo_b = (jnp.maximum(plo_v - L, 0) // jnp.int32(L))[0]
            phi_pad = jnp.minimum(phi_v + L, jnp.int32(N))
            phi_b = ((phi_pad + jnp.int32(L - 1)) // jnp.int32(L))[0]

            res = lax.fori_loop(plo_b, jnp.minimum(lo_b, phi_b), tgt_body, res)
            res = lax.fori_loop(jnp.maximum(lo_b + WB, plo_b), phi_b, tgt_body, res)
            mv = res[:SBLK]
            mi = res[SBLK:]

            osx = src_v[0, pl.ds(o, L)]
            osy = src_v[1, pl.ds(o, L)]
            osz = src_v[2, pl.ds(o, L)]
            for k in range(SBLK):
                gm = -_lane_bcast(plsc.cummax(-mv[k]), L - 1)
                cand = jnp.where(mv[k] == gm, mi[k], big_i)
                idx = -_lane_bcast(plsc.cummax(-cand), L - 1)
                gx = plsc.load_gather(tgt_v, [zero_i, idx])
                gy = plsc.load_gather(tgt_v, [one_i, idx])
                gz = plsc.load_gather(tgt_v, [two_i, idx])
                ox = _lane_bcast(osx, hb + k)
                oy = _lane_bcast(osy, hb + k)
                oz = _lane_bcast(osz, hb + k)
                dx = ox - gx
                dy = oy - gy
                dz = oz - gz
                dd = (dx * dx + dy * dy) + dz * dz
                svalid = (ox != 0.0) | (oy != 0.0) | (oz != 0.0)
                dacc = dacc + jnp.where(svalid, dd, zero_v)
        return dacc

    dacc = lax.fori_loop(0, NSV, grp_body, zero_v)

    out_v[0, :] = dacc
    out_v[1, :] = cta
    pltpu.sync_copy(out_v, out_hbm.at[wid])


def kernel(source_point_cloud, target_point_cloud):
    srcT = source_point_cloud.astype(jnp.float32)
    tgtT = target_point_cloud.astype(jnp.float32)

    def norm_sort(p):
        r = p.astype(jnp.bfloat16).astype(jnp.float32)
        n = jnp.sqrt(jnp.sum(r * r, axis=-1))
        order = jnp.argsort(n, axis=1)
        ps = jnp.take_along_axis(p, order[..., None], axis=1)
        ns = jnp.take_along_axis(n, order, axis=1)
        return jnp.concatenate([ps.transpose(0, 2, 1), ns[:, None, :]], axis=1)

    src4 = norm_sort(srcT)
    tgt4 = norm_sort(tgtT)
    src_chunks = (
        src4.reshape(B, 4, CHUNKS, CHUNK).transpose(0, 2, 1, 3).reshape(NW, 4, CHUNK)
    )
    part = _sc_knn_loss(src_chunks, tgt4)

    d2sum = part[:, 0, 0]
    cnt = part[:, 1, :].sum(-1)
    loss_b = d2sum.reshape(B, CHUNKS).sum(1) / (3.0 * cnt.reshape(B, CHUNKS).sum(1))
    return jnp.mean(loss_b)
